# baseline (speedup 1.0000x reference)
import functools

import jax
import jax.numpy as jnp
from jax.experimental import pallas as pl
from jax.experimental.pallas import tpu as pltpu

BSIZE = 32
ROWS_PER_TILE = 16

_DIMNUMS_CT_LHS = (((0,), (0,)), ((), ()))


def _row_kernel(idx_ref, na_ref, x_ref, w_ref, b_ref, o_ref, *, nbi):
    t = pl.program_id(0)
    w_img = x_ref.shape[2]
    nbj = w_img // BSIZE
    n_idx = idx_ref.shape[0]
    na = na_ref[0]
    block_row = t // (BSIZE // ROWS_PER_TILE)

    def scan_body(k, bits):
        valid = k < na
        rid = idx_ref[k, 0] * nbi + idx_ref[k, 1]
        hit = jnp.logical_and(valid, rid == block_row)
        return bits | jnp.where(hit, jnp.int32(1) << idx_ref[k, 2], jnp.int32(0))

    bits = jax.lax.fori_loop(0, n_idx, scan_body, jnp.int32(0))

    lane_blk = jax.lax.broadcasted_iota(jnp.int32, (1, w_img), 1) // BSIZE
    mv = jnp.zeros((1, w_img), jnp.float32)
    for j in range(nbj):
        m_j = (bits >> j) & 1
        mv = mv + jnp.where(lane_blk == j, m_j.astype(jnp.float32), 0.0)

    b_col = jnp.transpose(b_ref[...], (1, 0))
    for r in range(x_ref.shape[0]):
        q = jax.lax.dot_general(w_ref[...], x_ref[r], _DIMNUMS_CT_LHS,
                                preferred_element_type=jnp.float32)
        o_ref[r] = (q + b_col) * mv


def kernel(sbnet_x, active_block_indices, num_active, Wc, bc):
    n_batch, h, w, c = sbnet_x.shape
    oc = Wc.shape[-1]
    nbi = h // BSIZE

    na = jnp.reshape(jnp.asarray(num_active, jnp.int32), (1,))

    xt = jnp.transpose(sbnet_x, (0, 1, 3, 2)).reshape(n_batch * h, c, w)
    w2 = Wc.reshape(c, oc)
    b2 = bc.reshape(1, oc)

    out = pl.pallas_call(
        functools.partial(_row_kernel, nbi=nbi),
        grid_spec=pltpu.PrefetchScalarGridSpec(
            num_scalar_prefetch=2,
            grid=(n_batch * h // ROWS_PER_TILE,),
            in_specs=[
                pl.BlockSpec((ROWS_PER_TILE, c, w), lambda t, i_, n_: (t, 0, 0)),
                pl.BlockSpec((c, oc), lambda t, i_, n_: (0, 0)),
                pl.BlockSpec((1, oc), lambda t, i_, n_: (0, 0)),
            ],
            out_specs=pl.BlockSpec((ROWS_PER_TILE, oc, w),
                                   lambda t, i_, n_: (t, 0, 0)),
        ),
        out_shape=jax.ShapeDtypeStruct((n_batch * h, oc, w), sbnet_x.dtype),
    )(active_block_indices, na, xt, w2, b2)
    return out.reshape(n_batch, h, oc, w).transpose(0, 1, 3, 2)


# speedup vs baseline: 1.3595x; 1.3595x over previous
import functools

import jax
import jax.numpy as jnp
from jax.experimental import pallas as pl
from jax.experimental.pallas import tpu as pltpu

BSIZE = 32

_DIMNUMS_CT_LHS = (((0,), (0,)), ((), ()))


def _row_kernel(idx_ref, na_ref, x_ref, w_ref, b_ref, o_ref, *, nbi):
    t = pl.program_id(0)
    w_img = x_ref.shape[2]
    nbj = w_img // BSIZE
    n_idx = idx_ref.shape[0]
    na = na_ref[0]

    def scan_body(k, bits):
        valid = k < na
        rid = idx_ref[k, 0] * nbi + idx_ref[k, 1]
        hit = jnp.logical_and(valid, rid == t)
        return bits | jnp.where(hit, jnp.int32(1) << idx_ref[k, 2], jnp.int32(0))

    bits = jax.lax.fori_loop(0, n_idx, scan_body, jnp.int32(0))

    lane_blk = jax.lax.broadcasted_iota(jnp.int32, (1, w_img), 1) // BSIZE
    mv = jnp.zeros((1, w_img), jnp.float32)
    for j in range(nbj):
        m_j = (bits >> j) & 1
        mv = mv + jnp.where(lane_blk == j, m_j.astype(jnp.float32), 0.0)

    b_col = jnp.transpose(b_ref[...], (1, 0))
    for r in range(x_ref.shape[0]):
        q = jax.lax.dot_general(w_ref[...], x_ref[r], _DIMNUMS_CT_LHS,
                                preferred_element_type=jnp.float32)
        o_ref[r] = (q + b_col) * mv


def kernel(sbnet_x, active_block_indices, num_active, Wc, bc):
    n_batch, h, w, c = sbnet_x.shape
    oc = Wc.shape[-1]
    nbi = h // BSIZE

    na = jnp.reshape(jnp.asarray(num_active, jnp.int32), (1,))

    xt = jnp.transpose(sbnet_x, (0, 1, 3, 2)).reshape(n_batch * h, c, w)
    w2 = Wc.reshape(c, oc)
    b2 = bc.reshape(1, oc)

    out = pl.pallas_call(
        functools.partial(_row_kernel, nbi=nbi),
        grid_spec=pltpu.PrefetchScalarGridSpec(
            num_scalar_prefetch=2,
            grid=(n_batch * nbi,),
            in_specs=[
                pl.BlockSpec((BSIZE, c, w), lambda t, i_, n_: (t, 0, 0)),
                pl.BlockSpec((c, oc), lambda t, i_, n_: (0, 0)),
                pl.BlockSpec((1, oc), lambda t, i_, n_: (0, 0)),
            ],
            out_specs=pl.BlockSpec((BSIZE, oc, w), lambda t, i_, n_: (t, 0, 0)),
        ),
        out_shape=jax.ShapeDtypeStruct((n_batch * h, oc, w), sbnet_x.dtype),
        compiler_params=pltpu.CompilerParams(
            dimension_semantics=("parallel",),
        ),
    )(active_block_indices, na, xt, w2, b2)
    return out.reshape(n_batch, h, oc, w).transpose(0, 1, 3, 2)
